# baseline (speedup 1.0000x reference)
import jax
import jax.numpy as jnp
from jax import lax
from jax.experimental import pallas as pl
from jax.experimental.pallas import tpu as pltpu
from jax.experimental.pallas import tpu_sc as plsc

DEPTH = 1000
N = 16384
NC = 2
NS = 16
NW = NC * NS
L = 16

N_SC = 4096
N_TC = N - N_SC
ROWS_PER_W = N_SC // NW
CHUNK = 32
NCHUNK = ROWS_PER_W // CHUNK
TC_BLOCK = 512


def _onehot_sc_body(idx_hbm, zeros_hbm, out_hbm, idx_v, buf0, buf1, sem0,
                    sem1):
    wid = lax.axis_index("s") * NC + lax.axis_index("c")
    base = wid * ROWS_PER_W

    pltpu.sync_copy(idx_hbm.at[pl.ds(base, ROWS_PER_W)], idx_v)
    z0 = pltpu.async_copy(zeros_hbm, buf0, sem0)
    z1 = pltpu.async_copy(zeros_hbm, buf1, sem1)
    z0.wait()
    z1.wait()

    iota = lax.iota(jnp.int32, L)
    ones = jnp.ones((L,), jnp.float32)
    zvec = jnp.zeros((L,), jnp.float32)
    bufs = (buf0, buf1)
    sems = (sem0, sem1)
    copies = [None, None]
    prev_pos = [None, None]

    for c in range(NCHUNK):
        b = c & 1
        buf = bufs[b]
        if copies[b] is not None:
            copies[b].wait()
            for rows, cols in prev_pos[b]:
                plsc.store_scatter(buf, [rows, cols], zvec)
        pos = []
        for j in range(CHUNK // L):
            cols = idx_v[pl.ds(c * CHUNK + j * L, L)]
            rows = iota + (j * L)
            plsc.store_scatter(buf, [rows, cols], ones)
            pos.append((rows, cols))
        prev_pos[b] = pos
        copies[b] = pltpu.async_copy(
            buf, out_hbm.at[pl.ds(base + c * CHUNK, CHUNK)], sems[b])

    copies[0].wait()
    copies[1].wait()


_mesh = plsc.VectorSubcoreMesh(core_axis_name="c", subcore_axis_name="s")

_onehot_sc = pl.kernel(
    _onehot_sc_body,
    out_type=jax.ShapeDtypeStruct((N_SC, DEPTH), jnp.float32),
    mesh=_mesh,
    scratch_types=[
        pltpu.VMEM((ROWS_PER_W,), jnp.int32),
        pltpu.VMEM((CHUNK, DEPTH), jnp.float32),
        pltpu.VMEM((CHUNK, DEPTH), jnp.float32),
        pltpu.SemaphoreType.DMA,
        pltpu.SemaphoreType.DMA,
    ],
    compiler_params=pltpu.CompilerParams(
        use_tc_tiling_on_sc=True, needs_layout_passes=False),
)


def _onehot_tc_body(idx_ref, out_ref):
    idx = idx_ref[...]
    cols = lax.broadcasted_iota(jnp.int32, (TC_BLOCK, DEPTH), 1)
    out_ref[...] = (cols == idx).astype(jnp.float32)


_onehot_tc = pl.pallas_call(
    _onehot_tc_body,
    grid=(N_TC // TC_BLOCK,),
    in_specs=[pl.BlockSpec((TC_BLOCK, 1), lambda i: (i, 0))],
    out_specs=pl.BlockSpec((TC_BLOCK, DEPTH), lambda i: (i, 0)),
    out_shape=jax.ShapeDtypeStruct((N_TC, DEPTH), jnp.float32),
)


def kernel(inputs):
    idx = inputs.astype(jnp.int32)
    tc_part = _onehot_tc(idx[:N_TC])
    zeros = jnp.zeros((CHUNK, DEPTH), jnp.float32)
    sc_part = _onehot_sc(idx[N_TC:].reshape(-1), zeros)
    return jnp.concatenate([tc_part, sc_part], axis=0)


# speedup vs baseline: 2.8640x; 2.8640x over previous
import jax
import jax.numpy as jnp
from jax import lax
from jax.experimental import pallas as pl
from jax.experimental.pallas import tpu as pltpu
from jax.experimental.pallas import tpu_sc as plsc

DEPTH = 1000
N = 16384
NC = 2
NS = 16
NW = NC * NS
L = 16
COLS_PER_W = N // NW
RCH = 40
NCHUNK = DEPTH // RCH


def _scatter_chunk(buf, idx_v, r0, val):
    iota = lax.iota(jnp.int32, L)

    def body(t, carry):
        j0 = t * L
        v = idx_v[pl.ds(j0, L)]
        m = (v >= r0) & (v < r0 + RCH)
        plsc.store_scatter(buf, [v - r0, iota + j0], val, mask=m)
        return carry

    lax.fori_loop(0, COLS_PER_W // L, body, 0)


def _onehot_body(idx_hbm, zeros_hbm, out_hbm, idx_v, buf0, buf1, sem0, sem1):
    wid = lax.axis_index("s") * NC + lax.axis_index("c")
    c0 = wid * COLS_PER_W

    pltpu.sync_copy(idx_hbm.at[pl.ds(c0, COLS_PER_W)], idx_v)
    z0 = pltpu.async_copy(zeros_hbm, buf0, sem0)
    z1 = pltpu.async_copy(zeros_hbm, buf1, sem1)
    z0.wait()
    z1.wait()

    ones = jnp.ones((L,), jnp.float32)
    zvec = jnp.zeros((L,), jnp.float32)
    bufs = (buf0, buf1)
    sems = (sem0, sem1)
    copies = [None, None]

    for c in range(NCHUNK):
        b = c & 1
        buf = bufs[b]
        if copies[b] is not None:
            copies[b].wait()
            _scatter_chunk(buf, idx_v, (c - 2) * RCH, zvec)
        _scatter_chunk(buf, idx_v, c * RCH, ones)
        copies[b] = pltpu.async_copy(
            buf, out_hbm.at[pl.ds(c * RCH, RCH), pl.ds(c0, COLS_PER_W)],
            sems[b])

    copies[0].wait()
    copies[1].wait()


_mesh = plsc.VectorSubcoreMesh(core_axis_name="c", subcore_axis_name="s")

_onehot_t = pl.kernel(
    _onehot_body,
    out_type=jax.ShapeDtypeStruct((DEPTH, N), jnp.float32),
    mesh=_mesh,
    scratch_types=[
        pltpu.VMEM((COLS_PER_W,), jnp.int32),
        pltpu.VMEM((RCH, COLS_PER_W), jnp.float32),
        pltpu.VMEM((RCH, COLS_PER_W), jnp.float32),
        pltpu.SemaphoreType.DMA,
        pltpu.SemaphoreType.DMA,
    ],
    compiler_params=pltpu.CompilerParams(
        use_tc_tiling_on_sc=True, needs_layout_passes=False),
)


def kernel(inputs):
    idx = inputs.reshape(-1).astype(jnp.int32)
    zeros = jnp.zeros((RCH, COLS_PER_W), jnp.float32)
    return _onehot_t(idx, zeros).T
